# baseline (speedup 1.0000x reference)
import functools

import jax
import jax.numpy as jnp
from jax import lax
from jax.experimental import pallas as pl
from jax.experimental.pallas import tpu as pltpu
from jax.experimental.pallas import tpu_sc as plsc

B = 16384
D = 64
NC = 2
NS = 16
NW = NC * NS
BPW = B // NW
CH = 16
NCHUNK = BPW // CH
LG = 16


def _scalar(vec, lane_iota, e):
    return jnp.max(jnp.where(lane_iota == e, vec, jnp.int32(-1)))


def _body(xu_h, xv_h, uid_t, iid_t, o3, xu_v, xv_v, blk, stage, sem):
    wid = lax.axis_index("s") * NC + lax.axis_index("c")
    base = wid * BPW
    pltpu.sync_copy(xu_h.at[pl.ds(base, BPW)], xu_v)
    pltpu.sync_copy(xv_h.at[pl.ds(base, BPW)], xv_v)
    lane_iota = lax.iota(jnp.int32, 16)

    for t in range(2):
        idx_v = (xu_v, xv_v)[t]
        tab = (uid_t, iid_t)[t]

        def chunk(c, _):
            vec = idx_v[pl.ds(c * CH, CH)]
            rs = [_scalar(vec, lane_iota, e) for e in range(CH)]
            copies = []
            for e in range(CH):
                lam16 = pl.multiple_of((rs[e] >> 4) << 4, LG)
                copies.append(pltpu.async_copy(
                    tab.at[:, pl.ds(lam16, LG)],
                    blk.at[:, pl.ds(e * LG, LG)], sem))
            for cp in copies:
                cp.wait()
            for e in range(CH):
                col = jnp.broadcast_to(e * LG + (rs[e] & 15), (16,))
                ocol = jnp.broadcast_to(c * CH + e, (16,))
                for q in range(D // 16):
                    dvec = q * 16 + lane_iota
                    vals = plsc.load_gather(blk, [dvec, col])
                    plsc.store_scatter(stage, [dvec, ocol], vals)
            return ()

        lax.fori_loop(0, NCHUNK, chunk, (), unroll=False)
        pltpu.sync_copy(stage, o3.at[t, :, pl.ds(base, BPW)])


@jax.jit
def _lookup(xu, xv, uid_t, iid_t):
    mesh = plsc.VectorSubcoreMesh(core_axis_name="c", subcore_axis_name="s")
    f = functools.partial(
        pl.kernel,
        mesh=mesh,
        out_type=jax.ShapeDtypeStruct((2, D, B), jnp.float32),
        scratch_types=[
            pltpu.VMEM((BPW,), jnp.int32),
            pltpu.VMEM((BPW,), jnp.int32),
            pltpu.VMEM((D, CH * LG), jnp.float32),
            pltpu.VMEM((D, BPW), jnp.float32),
            pltpu.SemaphoreType.DMA,
        ],
        compiler_params=pltpu.CompilerParams(
            use_tc_tiling_on_sc=False, needs_layout_passes=False),
    )(_body)
    return f(xu, xv, uid_t, iid_t)


def kernel(x, uid_table, iid_table):
    xi = x.astype(jnp.int32)
    ut = uid_table.T
    it = iid_table.T
    o3 = _lookup(xi[:, 0], xi[:, 1], ut, it)
    return jnp.transpose(o3, (2, 0, 1))


# speedup vs baseline: 18.3569x; 18.3569x over previous
import functools

import jax
import jax.numpy as jnp
from jax import lax
from jax.experimental import pallas as pl
from jax.experimental.pallas import tpu as pltpu
from jax.experimental.pallas import tpu_sc as plsc

B = 16384
D = 64
NC = 2
NS = 16
NW = NC * NS
BPW = B // NW
CH = 16
NCHUNK = BPW // CH
TROWS = 8


def _scalar(vec, lane_iota, e):
    return jnp.max(jnp.where(lane_iota == e, vec, jnp.int32(-1)))


def _body(xu_h, xv_h, uid_tab, iid_tab, o3,
          xu_v, xv_v, tiles, stage, sem):
    wid = lax.axis_index("s") * NC + lax.axis_index("c")
    base = wid * BPW
    pltpu.sync_copy(xu_h.at[pl.ds(base, BPW)], xu_v)
    pltpu.sync_copy(xv_h.at[pl.ds(base, BPW)], xv_v)
    lane_iota = lax.iota(jnp.int32, 16)

    for t in range(2):
        idx_v = (xu_v, xv_v)[t]
        tab = (uid_tab, iid_tab)[t]

        def chunk(c, _):
            vec = idx_v[pl.ds(c * CH, CH)]
            rs = [_scalar(vec, lane_iota, e) for e in range(CH)]
            copies = []
            for e in range(CH):
                copies.append(pltpu.async_copy(
                    tab.at[pl.ds(rs[e] >> 3, 1)], tiles.at[pl.ds(e, 1)], sem))
            for cp in copies:
                cp.wait()
            for e in range(CH):
                su = rs[e] & 7
                col = jnp.broadcast_to(c * CH + e, (16,))
                for q in range(D // 16):
                    vals = tiles[e, su, pl.ds(16 * q, 16)]
                    plsc.store_scatter(stage, [16 * q + lane_iota, col], vals)
            return ()

        lax.fori_loop(0, NCHUNK, chunk, (), unroll=False)
        pltpu.sync_copy(stage, o3.at[t, :, pl.ds(base, BPW)])


@jax.jit
def _lookup(xu, xv, uid_t, iid_t):
    mesh = plsc.VectorSubcoreMesh(core_axis_name="c", subcore_axis_name="s")
    f = functools.partial(
        pl.kernel,
        mesh=mesh,
        out_type=jax.ShapeDtypeStruct((2, D, B), jnp.float32),
        scratch_types=[
            pltpu.VMEM((BPW,), jnp.int32),
            pltpu.VMEM((BPW,), jnp.int32),
            pltpu.VMEM((CH, TROWS, D), jnp.float32),
            pltpu.VMEM((D, BPW), jnp.float32),
            pltpu.SemaphoreType.DMA,
        ],
        compiler_params=pltpu.CompilerParams(needs_layout_passes=False),
    )(_body)
    return f(xu, xv, uid_t, iid_t)


def kernel(x, uid_table, iid_table):
    xi = x.astype(jnp.int32)
    ut = uid_table.reshape(125000, TROWS, D)
    it = iid_table[:1000000].reshape(125000, TROWS, D)
    o3 = _lookup(xi[:, 0], xi[:, 1], ut, it)
    return jnp.transpose(o3, (2, 0, 1))


# speedup vs baseline: 19.5807x; 1.0667x over previous
import functools

import jax
import jax.numpy as jnp
from jax import lax
from jax.experimental import pallas as pl
from jax.experimental.pallas import tpu as pltpu
from jax.experimental.pallas import tpu_sc as plsc

B = 16384
D = 64
NC = 2
NS = 16
NW = NC * NS
BPW = B // NW
CH = 16
NCHUNK = BPW // CH
TROWS = 8


def _scalar(vec, lane_iota, e):
    return jnp.max(jnp.where(lane_iota == e, vec, jnp.int32(-1)))


def _body(xu_h, xv_h, uid_tab, iid_tab, out,
          xu_v, xv_v, tiles_u, tiles_i, rows_c, sem):
    wid = lax.axis_index("s") * NC + lax.axis_index("c")
    base = wid * BPW
    pltpu.sync_copy(xu_h.at[pl.ds(base, BPW)], xu_v)
    pltpu.sync_copy(xv_h.at[pl.ds(base, BPW)], xv_v)
    lane_iota = lax.iota(jnp.int32, 16)

    def chunk(c, _):
        vec_u = xu_v[pl.ds(c * CH, CH)]
        vec_i = xv_v[pl.ds(c * CH, CH)]
        rus = [_scalar(vec_u, lane_iota, e) for e in range(CH)]
        ris = [_scalar(vec_i, lane_iota, e) for e in range(CH)]
        copies = []
        for e in range(CH):
            copies.append(pltpu.async_copy(
                uid_tab.at[pl.ds(rus[e] >> 3, 1)], tiles_u.at[pl.ds(e, 1)], sem))
            copies.append(pltpu.async_copy(
                iid_tab.at[pl.ds(ris[e] >> 3, 1)], tiles_i.at[pl.ds(e, 1)], sem))
        for cp in copies:
            cp.wait()
        for e in range(CH):
            su = rus[e] & 7
            si = ris[e] & 7
            for q in range(D // 16):
                rows_c[e, pl.ds(16 * q, 16)] = tiles_u[e, su, pl.ds(16 * q, 16)]
                rows_c[e, pl.ds(D + 16 * q, 16)] = tiles_i[e, si, pl.ds(16 * q, 16)]
        pltpu.sync_copy(rows_c, out.at[pl.ds(base + c * CH, CH)])
        return ()

    lax.fori_loop(0, NCHUNK, chunk, (), unroll=False)


@jax.jit
def _lookup(xu, xv, uid_t, iid_t):
    mesh = plsc.VectorSubcoreMesh(core_axis_name="c", subcore_axis_name="s")
    f = functools.partial(
        pl.kernel,
        mesh=mesh,
        out_type=jax.ShapeDtypeStruct((B, 2 * D), jnp.float32),
        scratch_types=[
            pltpu.VMEM((BPW,), jnp.int32),
            pltpu.VMEM((BPW,), jnp.int32),
            pltpu.VMEM((CH, TROWS, D), jnp.float32),
            pltpu.VMEM((CH, TROWS, D), jnp.float32),
            pltpu.VMEM((CH, 2 * D), jnp.float32),
            pltpu.SemaphoreType.DMA,
        ],
        compiler_params=pltpu.CompilerParams(needs_layout_passes=False),
    )(_body)
    return f(xu, xv, uid_t, iid_t)


def kernel(x, uid_table, iid_table):
    xi = x.astype(jnp.int32)
    ut = uid_table.reshape(125000, TROWS, D)
    it = iid_table[:1000000].reshape(125000, TROWS, D)
    o2 = _lookup(xi[:, 0], xi[:, 1], ut, it)
    return o2.reshape(B, 2, D)
